# baseline (device time: 103122 ns/iter reference)
import jax
import jax.numpy as jnp
from jax import lax
from jax.experimental import pallas as pl
from jax.experimental.pallas import tpu as pltpu

N_DEV = 16
N_MSG = 15
M_HALF = 128


def kernel(x, w_mat, scale_x, scale_w):
    m_per, k = x.shape
    k2, n_per = w_mat.shape
    assert k == k2 and m_per == 2 * M_HALF

    def body(x_ref, w_ref, sx_ref, sw_ref, out_ref,
             fwd_ref, bwd_ref, w8_ref,
             fs_sems, fr_sems, bs_sems, br_sems):
        my = lax.axis_index("i")
        left = lax.rem(my + N_DEV - 1, N_DEV)
        right = lax.rem(my + 1, N_DEV)

        scale = sx_ref[0] * sw_ref[0]

        x8 = x_ref[...].astype(jnp.float8_e4m3fn)

        def mk(buf_ref, s, ssems, rsems, dev):
            return pltpu.make_async_remote_copy(
                src_ref=buf_ref.at[pl.ds(s * M_HALF, M_HALF), :],
                dst_ref=buf_ref.at[pl.ds((s + 2) * M_HALF, M_HALF), :],
                send_sem=ssems.at[s],
                recv_sem=rsems.at[s],
                device_id=(dev,),
                device_id_type=pl.DeviceIdType.MESH,
            )

        fwd_rdmas = [mk(fwd_ref, s, fs_sems, fr_sems, right) for s in range(N_MSG)]
        bwd_rdmas = [mk(bwd_ref, s, bs_sems, br_sems, left) for s in range(N_MSG)]

        fwd_ref[pl.ds(0, M_HALF), :] = x8[:M_HALF]
        bwd_ref[pl.ds(0, M_HALF), :] = x8[M_HALF:]
        fwd_ref[pl.ds(M_HALF, M_HALF), :] = x8[M_HALF:]
        bwd_ref[pl.ds(M_HALF, M_HALF), :] = x8[:M_HALF]
        w8_ref[...] = w_ref[...].astype(jnp.float8_e4m3fn)
        acc = jnp.dot(x8, w8_ref[...], preferred_element_type=jnp.float32)
        out_ref[pl.ds(my * m_per, m_per), :] = acc * scale

        barrier_sem = pltpu.get_barrier_semaphore()
        for nbr in (left, right):
            pl.semaphore_signal(
                barrier_sem, inc=1,
                device_id=(nbr,), device_id_type=pl.DeviceIdType.MESH,
            )
        pl.semaphore_wait(barrier_sem, 2)

        fwd_rdmas[0].start()
        bwd_rdmas[0].start()
        fwd_rdmas[1].start()
        bwd_rdmas[1].start()

        origin8 = lax.rem(my + N_DEV // 2, N_DEV)

        for s in range(N_MSG):
            fwd_rdmas[s].wait_recv()
            if s + 2 < N_MSG:
                fwd_rdmas[s + 2].start()
            if s == N_MSG - 1:
                acc = jnp.dot(fwd_ref[pl.ds(16 * M_HALF, M_HALF), :],
                              w8_ref[...], preferred_element_type=jnp.float32)
                out_ref[pl.ds(origin8 * m_per, M_HALF), :] = acc * scale
            bwd_rdmas[s].wait_recv()
            if s + 2 < N_MSG:
                bwd_rdmas[s + 2].start()

            if s < N_MSG - 1:
                d = s // 2 + 1
                origin_f = lax.rem(my + N_DEV - d, N_DEV)
                origin_b = lax.rem(my + d, N_DEV)
                f_off = origin_f * m_per + (0 if s % 2 == 0 else M_HALF)
                b_off = origin_b * m_per + (M_HALF if s % 2 == 0 else 0)
                acc = jnp.dot(fwd_ref[pl.ds((s + 2) * M_HALF, M_HALF), :],
                              w8_ref[...], preferred_element_type=jnp.float32)
                out_ref[pl.ds(f_off, M_HALF), :] = acc * scale
                acc = jnp.dot(bwd_ref[pl.ds((s + 2) * M_HALF, M_HALF), :],
                              w8_ref[...], preferred_element_type=jnp.float32)
                out_ref[pl.ds(b_off, M_HALF), :] = acc * scale

        acc = jnp.dot(bwd_ref[pl.ds(16 * M_HALF, M_HALF), :], w8_ref[...],
                      preferred_element_type=jnp.float32)
        out_ref[pl.ds(origin8 * m_per + M_HALF, M_HALF), :] = acc * scale

        for r in fwd_rdmas:
            r.wait_send()
        for r in bwd_rdmas:
            r.wait_send()

    n_slots = N_MSG + 2
    return pl.pallas_call(
        body,
        out_shape=jax.ShapeDtypeStruct((N_DEV * m_per, n_per), jnp.float32),
        in_specs=[
            pl.BlockSpec(memory_space=pltpu.VMEM),
            pl.BlockSpec(memory_space=pltpu.VMEM),
            pl.BlockSpec(memory_space=pltpu.SMEM),
            pl.BlockSpec(memory_space=pltpu.SMEM),
        ],
        out_specs=pl.BlockSpec(memory_space=pltpu.VMEM),
        scratch_shapes=[
            pltpu.VMEM((n_slots * M_HALF, k), jnp.float8_e4m3fn),
            pltpu.VMEM((n_slots * M_HALF, k), jnp.float8_e4m3fn),
            pltpu.VMEM((k, n_per), jnp.float8_e4m3fn),
            pltpu.SemaphoreType.DMA((N_MSG,)),
            pltpu.SemaphoreType.DMA((N_MSG,)),
            pltpu.SemaphoreType.DMA((N_MSG,)),
            pltpu.SemaphoreType.DMA((N_MSG,)),
        ],
        compiler_params=pltpu.CompilerParams(collective_id=0),
    )(x, w_mat, scale_x, scale_w)
